# baseline (speedup 1.0000x reference)
import dataclasses
import functools

import jax
import jax.numpy as jnp
from jax import lax
from jax.experimental import pallas as pl
from jax.experimental.pallas import tpu as pltpu
from jax.experimental.pallas import tpu_sc as plsc

NUM_CLASSES = 1000
MIN_W = 0.1
B = 4096
M = 200
L = 16
NFULL = 12
TAIL_OFF = 184
NW = 32
RPW = B // NW
RPB = 8
NBLK = RPW // RPB
NBUF = 4

_mesh = plsc.VectorSubcoreMesh(core_axis_name="c", subcore_axis_name="s")

_cp = pltpu.CompilerParams()
if "needs_layout_passes" in pltpu.CompilerParams.__dataclass_fields__:
    _cp = dataclasses.replace(_cp, needs_layout_passes=False)


@jax.jit
def _sc_histogram(labels, weights, recip):
    @functools.partial(
        pl.kernel,
        mesh=_mesh,
        compiler_params=_cp,
        out_type=jax.ShapeDtypeStruct((B, NUM_CLASSES), jnp.float32),
        scratch_types=[
            pltpu.VMEM((RPW, M), jnp.int32),
            pltpu.VMEM((RPW, M), jnp.float32),
            pltpu.VMEM((NBUF, RPB, NUM_CLASSES), jnp.float32),
            pltpu.VMEM((256,), jnp.float32),
            pltpu.SemaphoreType.DMA,
            pltpu.SemaphoreType.DMA,
            pltpu.SemaphoreType.DMA,
            pltpu.SemaphoreType.DMA,
            pltpu.SemaphoreType.DMA,
        ],
    )
    def k(lab_hbm, w_hbm, recip_hbm, out_hbm, lab_v, w_v, slabs_v, recip_v,
          sem0, sem1, sem2, sem3, sem_in):
        wid = lax.axis_index("s") * 2 + lax.axis_index("c")
        base = wid * RPW
        cp_lab = pltpu.make_async_copy(lab_hbm.at[pl.ds(base, RPW)], lab_v,
                                       sem_in)
        cp_w = pltpu.make_async_copy(w_hbm.at[pl.ds(base, RPW)], w_v, sem_in)
        cp_r = pltpu.make_async_copy(recip_hbm, recip_v, sem_in)
        cp_lab.start()
        cp_w.start()
        cp_r.start()

        sems = (sem0, sem1, sem2, sem3)
        zeros = jnp.zeros((L,), jnp.float32)
        tail_mask = lax.iota(jnp.int32, L) >= (NFULL * L - TAIL_OFF)

        for j in range(NBUF):
            slab = slabs_v.at[j]

            @pl.loop(0, (NUM_CLASSES // L) * L, step=4 * L)
            def _(i):
                for s in range(RPB):
                    for u in range(4):
                        slab[s, pl.ds(i + u * L, L)] = zeros
            for s in range(RPB):
                slab[s, pl.ds(NUM_CLASSES - L, L)] = zeros

        cp_lab.wait()
        cp_w.wait()
        cp_r.wait()

        def do_row(slab, r, s):
            ws = [w_v[r, pl.ds(c * L, L)] for c in range(NFULL)]
            wt = w_v[r, pl.ds(TAIL_OFF, L)]
            labs = [lab_v[r, pl.ds(c * L, L)] for c in range(NFULL)]
            labt = lab_v[r, pl.ds(TAIL_OFF, L)]
            masks = [w >= MIN_W for w in ws]
            mt = (wt >= MIN_W) & tail_mask
            cnt = plsc.all_reduce_population_count(mt)
            for m in masks:
                cnt = cnt + plsc.all_reduce_population_count(m)
            inv = plsc.load_gather(recip_v, [cnt])
            tiny_inv = 1e-10 * inv
            attns = [ws[c] * jnp.where(masks[c], inv, tiny_inv)
                     for c in range(NFULL)]
            attnt = wt * jnp.where(mt, inv, tiny_inv)
            svec = jnp.full((L,), s, jnp.int32)
            for c in range(NFULL):
                plsc.addupdate_scatter(slab, [svec, labs[c]], attns[c])
            plsc.addupdate_scatter(slab, [svec, labt], attnt, mask=tail_mask)

        def unzero_row(slab, r, s):
            labs = [lab_v[r, pl.ds(c * L, L)] for c in range(NFULL)]
            labt = lab_v[r, pl.ds(TAIL_OFF, L)]
            svec = jnp.full((L,), s, jnp.int32)
            for c in range(NFULL):
                plsc.store_scatter(slab, [svec, labs[c]], zeros)
            plsc.store_scatter(slab, [svec, labt], zeros)

        @pl.loop(0, NBLK, step=NBUF)
        def _(rb):
            for j in range(NBUF):
                rbx = rb + j
                slab, sem = slabs_v.at[j], sems[j]

                @pl.when(rbx >= NBUF)
                def _():
                    ob = rbx - NBUF
                    dst = out_hbm.at[pl.ds(base + ob * RPB, RPB)]
                    pltpu.make_async_copy(slab, dst, sem).wait()
                    for s in range(RPB):
                        for i in range(0, NUM_CLASSES - L, L):
                            slab[s, pl.ds(i, L)] = zeros
                        slab[s, pl.ds(NUM_CLASSES - L, L)] = zeros

                for s in range(RPB):
                    do_row(slab, rbx * RPB + s, s)

                dst = out_hbm.at[pl.ds(base + rbx * RPB, RPB)]
                pltpu.make_async_copy(slab, dst, sem).start()

        for j in range(NBUF):
            rbx = NBLK - NBUF + j
            dst = out_hbm.at[pl.ds(base + rbx * RPB, RPB)]
            pltpu.make_async_copy(slabs_v.at[j], dst, sems[j]).wait()

    return k(labels, weights, recip)


def kernel(input_embeddings, memories_labels, memories_embeddings,
           memories_weights):
    labels = memories_labels.astype(jnp.int32)
    recip = 1.0 / jnp.maximum(jnp.arange(256, dtype=jnp.float32), 1.0)
    return _sc_histogram(labels, memories_weights, recip)


# speedup vs baseline: 1.2622x; 1.2622x over previous
import dataclasses
import functools

import jax
import jax.numpy as jnp
from jax import lax
from jax.experimental import pallas as pl
from jax.experimental.pallas import tpu as pltpu
from jax.experimental.pallas import tpu_sc as plsc

NUM_CLASSES = 1000
MIN_W = 0.1
B = 4096
M = 200
L = 16
NFULL = 12
TAIL_OFF = 184
NW = 32
RPW = B // NW
RPB = 8
NBLK = RPW // RPB
NBUF = 2

_mesh = plsc.VectorSubcoreMesh(core_axis_name="c", subcore_axis_name="s")

_cp = pltpu.CompilerParams()
if "needs_layout_passes" in pltpu.CompilerParams.__dataclass_fields__:
    _cp = dataclasses.replace(_cp, needs_layout_passes=False)


@jax.jit
def _sc_histogram(labels, weights, recip):
    @functools.partial(
        pl.kernel,
        mesh=_mesh,
        compiler_params=_cp,
        out_type=jax.ShapeDtypeStruct((B, NUM_CLASSES), jnp.float32),
        scratch_types=[
            pltpu.VMEM((RPW, M), jnp.int32),
            pltpu.VMEM((RPW, M), jnp.float32),
            pltpu.VMEM((NBUF, RPB, NUM_CLASSES), jnp.float32),
            pltpu.VMEM((256,), jnp.float32),
            pltpu.SemaphoreType.DMA,
            pltpu.SemaphoreType.DMA,
            pltpu.SemaphoreType.DMA,
        ],
    )
    def k(lab_hbm, w_hbm, recip_hbm, out_hbm, lab_v, w_v, slabs_v, recip_v,
          sem0, sem1, sem_in):
        wid = lax.axis_index("s") * 2 + lax.axis_index("c")
        base = wid * RPW
        cp_lab = pltpu.make_async_copy(lab_hbm.at[pl.ds(base, RPW)], lab_v,
                                       sem_in)
        cp_w = pltpu.make_async_copy(w_hbm.at[pl.ds(base, RPW)], w_v, sem_in)
        cp_r = pltpu.make_async_copy(recip_hbm, recip_v, sem_in)
        cp_lab.start()
        cp_w.start()
        cp_r.start()

        sems = (sem0, sem1)
        zeros = jnp.zeros((L,), jnp.float32)
        tail_mask = lax.iota(jnp.int32, L) >= (NFULL * L - TAIL_OFF)

        for j in range(NBUF):
            slab = slabs_v.at[j]

            @pl.loop(0, (NUM_CLASSES // L) * L, step=4 * L)
            def _(i):
                for s in range(RPB):
                    for u in range(4):
                        slab[s, pl.ds(i + u * L, L)] = zeros
            for s in range(RPB):
                slab[s, pl.ds(NUM_CLASSES - L, L)] = zeros

        cp_lab.wait()
        cp_w.wait()
        cp_r.wait()

        def do_row(slab, r, s):
            ws = [w_v[r, pl.ds(c * L, L)] for c in range(NFULL)]
            wt = w_v[r, pl.ds(TAIL_OFF, L)]
            labs = [lab_v[r, pl.ds(c * L, L)] for c in range(NFULL)]
            labt = lab_v[r, pl.ds(TAIL_OFF, L)]
            masks = [w >= MIN_W for w in ws]
            mt = (wt >= MIN_W) & tail_mask
            ones = jnp.ones((L,), jnp.int32)
            zeros_i = jnp.zeros((L,), jnp.int32)
            cntv = jnp.where(mt, ones, zeros_i)
            for m in masks:
                cntv = cntv + jnp.where(m, ones, zeros_i)
            cnt = jnp.broadcast_to(jnp.sum(cntv), (L,))
            inv = plsc.load_gather(recip_v, [cnt])
            tiny_inv = 1e-10 * inv
            attns = [ws[c] * jnp.where(masks[c], inv, tiny_inv)
                     for c in range(NFULL)]
            attnt = wt * jnp.where(mt, inv, tiny_inv)
            svec = jnp.full((L,), s, jnp.int32)
            for c in range(NFULL):
                plsc.addupdate_scatter(slab, [svec, labs[c]], attns[c])
            plsc.addupdate_scatter(slab, [svec, labt], attnt, mask=tail_mask)

        def unzero_row(slab, r, s):
            labs = [lab_v[r, pl.ds(c * L, L)] for c in range(NFULL)]
            labt = lab_v[r, pl.ds(TAIL_OFF, L)]
            svec = jnp.full((L,), s, jnp.int32)
            for c in range(NFULL):
                plsc.store_scatter(slab, [svec, labs[c]], zeros)
            plsc.store_scatter(slab, [svec, labt], zeros)

        @pl.loop(0, NBLK, step=NBUF)
        def _(rb):
            for j in range(NBUF):
                rbx = rb + j
                slab, sem = slabs_v.at[j], sems[j]

                @pl.when(rbx >= NBUF)
                def _():
                    ob = rbx - NBUF
                    dst = out_hbm.at[pl.ds(base + ob * RPB, RPB)]
                    pltpu.make_async_copy(slab, dst, sem).wait()

                    @pl.loop(0, (NUM_CLASSES // L) * L, step=4 * L)
                    def _(i):
                        for s in range(RPB):
                            for u in range(4):
                                slab[s, pl.ds(i + u * L, L)] = zeros
                    for s in range(RPB):
                        slab[s, pl.ds(NUM_CLASSES - L, L)] = zeros

                for s in range(RPB):
                    do_row(slab, rbx * RPB + s, s)

                dst = out_hbm.at[pl.ds(base + rbx * RPB, RPB)]
                pltpu.make_async_copy(slab, dst, sem).start()

        for j in range(NBUF):
            rbx = NBLK - NBUF + j
            dst = out_hbm.at[pl.ds(base + rbx * RPB, RPB)]
            pltpu.make_async_copy(slabs_v.at[j], dst, sems[j]).wait()

    return k(labels, weights, recip)


def kernel(input_embeddings, memories_labels, memories_embeddings,
           memories_weights):
    labels = memories_labels.astype(jnp.int32)
    recip = 1.0 / jnp.maximum(jnp.arange(256, dtype=jnp.float32), 1.0)
    return _sc_histogram(labels, memories_weights, recip)


# speedup vs baseline: 1.2767x; 1.0115x over previous
import dataclasses
import functools

import jax
import jax.numpy as jnp
from jax import lax
from jax.experimental import pallas as pl
from jax.experimental.pallas import tpu as pltpu
from jax.experimental.pallas import tpu_sc as plsc

NUM_CLASSES = 1000
MIN_W = 0.1
B = 4096
M = 200
L = 16
NFULL = 12
TAIL_OFF = 184
NW = 32
RPW = B // NW
RPB = 8
NBLK = RPW // RPB
NBUF = 2

_mesh = plsc.VectorSubcoreMesh(core_axis_name="c", subcore_axis_name="s")

_cp = pltpu.CompilerParams()
if "needs_layout_passes" in pltpu.CompilerParams.__dataclass_fields__:
    _cp = dataclasses.replace(_cp, needs_layout_passes=False)


@jax.jit
def _sc_histogram(labels, weights, recip):
    @functools.partial(
        pl.kernel,
        mesh=_mesh,
        compiler_params=_cp,
        out_type=jax.ShapeDtypeStruct((B, NUM_CLASSES), jnp.float32),
        scratch_types=[
            pltpu.VMEM((RPW, M), jnp.int32),
            pltpu.VMEM((RPW, M), jnp.float32),
            pltpu.VMEM((NBUF, RPB, NUM_CLASSES), jnp.float32),
            pltpu.VMEM((256,), jnp.float32),
            pltpu.SemaphoreType.DMA,
            pltpu.SemaphoreType.DMA,
            pltpu.SemaphoreType.DMA,
        ],
    )
    def k(lab_hbm, w_hbm, recip_hbm, out_hbm, lab_v, w_v, slabs_v, recip_v,
          sem0, sem1, sem_in):
        wid = lax.axis_index("s") * 2 + lax.axis_index("c")
        base = wid * RPW
        cp_lab = pltpu.make_async_copy(lab_hbm.at[pl.ds(base, RPW)], lab_v,
                                       sem_in)
        cp_w = pltpu.make_async_copy(w_hbm.at[pl.ds(base, RPW)], w_v, sem_in)
        cp_r = pltpu.make_async_copy(recip_hbm, recip_v, sem_in)
        cp_lab.start()
        cp_w.start()
        cp_r.start()

        sems = (sem0, sem1)
        zeros = jnp.zeros((L,), jnp.float32)
        tail_mask = lax.iota(jnp.int32, L) >= (NFULL * L - TAIL_OFF)

        for j in range(NBUF):
            slab = slabs_v.at[j]

            @pl.loop(0, (NUM_CLASSES // L) * L, step=4 * L)
            def _(i):
                for s in range(RPB):
                    for u in range(4):
                        slab[s, pl.ds(i + u * L, L)] = zeros
            for s in range(RPB):
                slab[s, pl.ds(NUM_CLASSES - L, L)] = zeros

        cp_lab.wait()
        cp_w.wait()
        cp_r.wait()

        def do_row(slab, r, s):
            ws = [w_v[r, pl.ds(c * L, L)] for c in range(NFULL)]
            wt = w_v[r, pl.ds(TAIL_OFF, L)]
            labs = [lab_v[r, pl.ds(c * L, L)] for c in range(NFULL)]
            labt = lab_v[r, pl.ds(TAIL_OFF, L)]
            masks = [w >= MIN_W for w in ws]
            mt = (wt >= MIN_W) & tail_mask
            ones = jnp.ones((L,), jnp.int32)
            zeros_i = jnp.zeros((L,), jnp.int32)
            cntv = jnp.where(mt, ones, zeros_i)
            for m in masks:
                cntv = cntv + jnp.where(m, ones, zeros_i)
            cnt = jnp.broadcast_to(jnp.sum(cntv), (L,))
            inv = plsc.load_gather(recip_v, [cnt])
            tiny_inv = 1e-10 * inv
            attns = [ws[c] * jnp.where(masks[c], inv, tiny_inv)
                     for c in range(NFULL)]
            attnt = wt * jnp.where(mt, inv, tiny_inv)
            svec = jnp.full((L,), s, jnp.int32)
            for c in range(NFULL):
                plsc.addupdate_scatter(slab, [svec, labs[c]], attns[c])
            plsc.addupdate_scatter(slab, [svec, labt], attnt, mask=tail_mask)

        def unzero_row(slab, r, s):
            labs = [lab_v[r, pl.ds(c * L, L)] for c in range(NFULL)]
            labt = lab_v[r, pl.ds(TAIL_OFF, L)]
            svec = jnp.full((L,), s, jnp.int32)
            for c in range(NFULL):
                plsc.store_scatter(slab, [svec, labs[c]], zeros)
            plsc.store_scatter(slab, [svec, labt], zeros)

        @pl.loop(0, NBLK, step=NBUF)
        def _(rb):
            for j in range(NBUF):
                rbx = rb + j
                slab, sem = slabs_v.at[j], sems[j]

                @pl.when(rbx >= NBUF)
                def _():
                    ob = rbx - NBUF
                    dst = out_hbm.at[pl.ds(base + ob * RPB, RPB)]
                    pltpu.make_async_copy(slab, dst, sem).wait()

                    @pl.loop(0, (NUM_CLASSES // L) * L, step=4 * L)
                    def _(i):
                        for s in range(RPB):
                            for u in range(4):
                                slab[s, pl.ds(i + u * L, L)] = zeros
                    for s in range(RPB):
                        slab[s, pl.ds(NUM_CLASSES - L, L)] = zeros

                @pl.loop(0, RPB)
                def _(s):
                    do_row(slab, rbx * RPB + s, s)

                dst = out_hbm.at[pl.ds(base + rbx * RPB, RPB)]
                pltpu.make_async_copy(slab, dst, sem).start()

        for j in range(NBUF):
            rbx = NBLK - NBUF + j
            dst = out_hbm.at[pl.ds(base + rbx * RPB, RPB)]
            pltpu.make_async_copy(slabs_v.at[j], dst, sems[j]).wait()

    return k(labels, weights, recip)


def kernel(input_embeddings, memories_labels, memories_embeddings,
           memories_weights):
    labels = memories_labels.astype(jnp.int32)
    recip = 1.0 / jnp.maximum(jnp.arange(256, dtype=jnp.float32), 1.0)
    return _sc_histogram(labels, memories_weights, recip)
